# Initial kernel scaffold; baseline (speedup 1.0000x reference)
import jax
import jax.numpy as jnp
from jax.experimental import pallas as pl


def kernel(inp, norm_w, W_qkv, mem_kv, k_pos, v_pos, Wk_c, bk_c, Wv_c, bv_c, W_comb, b_comb, W_out):
    raise NotImplementedError("write your pallas kernel here")


# speedup vs baseline: 3.1438x; 3.1438x over previous
from functools import partial
from math import ceil

import jax
import jax.numpy as jnp
from jax.experimental import pallas as pl

_DIM = 768
_DIM_HEAD = 64
_HEADS = 12
_WINDOW = 32
_CBLOCK = 32
_SBLOCK = 32
_NSEL = 2
_NMEM = 4
_SEQ = 2048
_DIM_INNER = _DIM_HEAD * _HEADS

_QCHUNK = 256
_KCHUNK = 256
_NQ = _SEQ // _QCHUNK
_NC = _SEQ // _CBLOCK
_CKP = 128
_GPAD = 128

_MASKVAL = -jnp.finfo(jnp.float32).max
_NEGFILL = -3.0e38
_MINIT = -1.0e38
_EPS = jnp.finfo(jnp.float32).eps
_SCALE = _DIM_HEAD ** -0.5

_f32 = jnp.float32


def _dot(a, b):
    return jnp.dot(a, b, preferred_element_type=_f32)


def _dot_nt(a, b):
    return jax.lax.dot_general(a, b, (((1,), (1,)), ((), ())),
                               preferred_element_type=_f32)


def _pre_kernel(inp_ref, nw_ref, wqkv_ref, wcomb_ref, bcomb_ref,
                qkv_ref, comb_ref):
    x = inp_ref[...]
    xn = x * jax.lax.rsqrt(jnp.mean(x * x, axis=-1, keepdims=True) + _EPS)
    xn = xn * nw_ref[...]
    qkv_ref[...] = _dot(xn, wqkv_ref[...])
    comb_ref[...] = jax.nn.sigmoid(_dot(xn, wcomb_ref[...]) + bcomb_ref[...])


def _comp_kernel(k_ref, v_ref, kpos_ref, vpos_ref, wk_ref, wv_ref,
                 bk_ref, bv_ref, ck_ref, cv_ref):
    acc_k = jnp.zeros((_NC, _DIM_HEAD), _f32)
    acc_v = jnp.zeros((_NC, _DIM_HEAD), _f32)
    for t in range(_CBLOCK):
        kk_t = k_ref[0, :, t, :] + kpos_ref[0, t, :][None]
        vv_t = v_ref[0, :, t, :] + vpos_ref[0, t, :][None]
        acc_k = acc_k + _dot(kk_t, wk_ref[0, t])
        acc_v = acc_v + _dot(vv_t, wv_ref[0, t])
    ck_ref[0] = acc_k + bk_ref[0]
    cv_ref[0] = acc_v + bv_ref[0]


def _attn_kernel(q_ref, k_ref, v_ref, ck_ref, cv_ref, comb_ref, out_ref):
    h = pl.program_id(0)
    qi = pl.program_id(1)
    q = q_ref[0] * _SCALE
    rows = qi * _QCHUNK + jax.lax.broadcasted_iota(jnp.int32, (_QCHUNK, 1), 0)
    colsj = jax.lax.broadcasted_iota(jnp.int32, (1, _CKP), 1)

    csim = _dot_nt(q, ck_ref[0])
    ck_seq = jnp.where(colsj < _NMEM, -1, (colsj - _NMEM + 1) * _CBLOCK - 1)
    cmask = (ck_seq < rows) | (colsj >= _NMEM + _NC)
    csim = jnp.where(cmask, _MASKVAL, csim)
    cm = jnp.max(csim, axis=-1, keepdims=True)
    cp = jnp.exp(csim - cm)
    cl = jnp.sum(cp, axis=-1, keepdims=True)
    cmp_out = _dot(cp, cv_ref[0]) / cl

    imp = jnp.where((colsj >= _NMEM) & (colsj < _NMEM + _NC), csim, -jnp.inf)
    big = jnp.int32(1 << 20)
    m1 = jnp.max(imp, axis=-1, keepdims=True)
    a1 = jnp.min(jnp.where(imp == m1, colsj, big), axis=-1, keepdims=True)
    imp2 = jnp.where(colsj == a1, -jnp.inf, imp)
    m2 = jnp.max(imp2, axis=-1, keepdims=True)
    a2 = jnp.min(jnp.where(imp2 == m2, colsj, big), axis=-1, keepdims=True)
    sel0 = a1 - _NMEM
    sel1 = a2 - _NMEM
    v0 = m1 > _MASKVAL
    v1 = m2 > _MASKVAL
    allmask = v0 & v1
    e0 = (~v0) | allmask
    e1 = (~v1) | allmask

    def body(c, carry):
        mf, lf, af, ml, ll, al = carry
        kc = k_ref[0, pl.ds(c * _KCHUNK, _KCHUNK), :]
        vc = v_ref[0, pl.ds(c * _KCHUNK, _KCHUNK), :]
        s = _dot_nt(q, kc)
        colg = c * _KCHUNK + jax.lax.broadcasted_iota(jnp.int32, (1, _KCHUNK), 1)
        w = colg // _SBLOCK
        fallow = ((w == sel0) & e0) | ((w == sel1) & e1)
        sf = jnp.where(fallow, jnp.where(allmask, 0.0, s), _NEGFILL)
        lallow = (colg <= rows) & (rows - colg <= _WINDOW)
        sl = jnp.where(lallow, s, _NEGFILL)

        mf2 = jnp.maximum(mf, jnp.max(sf, axis=-1, keepdims=True))
        alf = jnp.exp(mf - mf2)
        pf = jnp.exp(sf - mf2)
        lf = lf * alf + jnp.sum(pf, axis=-1, keepdims=True)
        af = af * alf + _dot(pf, vc)

        ml2 = jnp.maximum(ml, jnp.max(sl, axis=-1, keepdims=True))
        all_ = jnp.exp(ml - ml2)
        pll = jnp.exp(sl - ml2)
        ll = ll * all_ + jnp.sum(pll, axis=-1, keepdims=True)
        al = al * all_ + _dot(pll, vc)
        return (mf2, lf, af, ml2, ll, al)

    mz = jnp.full((_QCHUNK, 1), _MINIT, _f32)
    lz = jnp.zeros((_QCHUNK, 1), _f32)
    az = jnp.zeros((_QCHUNK, _DIM_HEAD), _f32)
    _, lf, af, _, ll, al = jax.lax.fori_loop(
        0, _SEQ // _KCHUNK, body, (mz, lz, az, mz, lz, az))
    fine_out = af / lf
    local_out = al / ll

    comb = comb_ref[...]
    z = jnp.zeros_like(comb)
    g0 = jnp.sum(jnp.where(colsj == 3 * h, comb, z), axis=-1, keepdims=True)
    g1 = jnp.sum(jnp.where(colsj == 3 * h + 1, comb, z), axis=-1, keepdims=True)
    g2 = jnp.sum(jnp.where(colsj == 3 * h + 2, comb, z), axis=-1, keepdims=True)
    out_ref[0] = g0 * cmp_out + g1 * fine_out + g2 * local_out


def _proj_kernel(x_ref, w_ref, o_ref):
    o_ref[...] = _dot(x_ref[...], w_ref[...])


def kernel(inp, norm_w, W_qkv, mem_kv, k_pos, v_pos, Wk_c, bk_c, Wv_c, bv_c,
           W_comb, b_comb, W_out):
    x = inp[0]

    wqkv_t = W_qkv.T
    wcomb_t = jnp.pad(W_comb.T, ((0, 0), (0, _GPAD - 3 * _HEADS)))
    bcomb = jnp.pad(b_comb, (0, _GPAD - 3 * _HEADS))[None]

    qkv, comb = pl.pallas_call(
        _pre_kernel,
        grid=(_NQ,),
        in_specs=[
            pl.BlockSpec((_QCHUNK, _DIM), lambda i: (i, 0)),
            pl.BlockSpec((1, _DIM), lambda i: (0, 0)),
            pl.BlockSpec((_DIM, 3 * _DIM_INNER), lambda i: (0, 0)),
            pl.BlockSpec((_DIM, _GPAD), lambda i: (0, 0)),
            pl.BlockSpec((1, _GPAD), lambda i: (0, 0)),
        ],
        out_specs=[
            pl.BlockSpec((_QCHUNK, 3 * _DIM_INNER), lambda i: (i, 0)),
            pl.BlockSpec((_QCHUNK, _GPAD), lambda i: (i, 0)),
        ],
        out_shape=[
            jax.ShapeDtypeStruct((_SEQ, 3 * _DIM_INNER), _f32),
            jax.ShapeDtypeStruct((_SEQ, _GPAD), _f32),
        ],
    )(x, norm_w[None], wqkv_t, wcomb_t, bcomb)

    q = qkv[:, :_DIM_INNER].reshape(_SEQ, _HEADS, _DIM_HEAD).transpose(1, 0, 2)
    k = qkv[:, _DIM_INNER:2 * _DIM_INNER].reshape(_SEQ, _HEADS, _DIM_HEAD).transpose(1, 0, 2)
    v = qkv[:, 2 * _DIM_INNER:].reshape(_SEQ, _HEADS, _DIM_HEAD).transpose(1, 0, 2)

    k4 = k.reshape(_HEADS, _NC, _CBLOCK, _DIM_HEAD)
    v4 = v.reshape(_HEADS, _NC, _CBLOCK, _DIM_HEAD)
    wk_f = Wk_c.reshape(_HEADS, _DIM_HEAD, _DIM_HEAD, _CBLOCK).transpose(0, 3, 2, 1)
    wv_f = Wv_c.reshape(_HEADS, _DIM_HEAD, _DIM_HEAD, _CBLOCK).transpose(0, 3, 2, 1)
    bk2 = bk_c.reshape(_HEADS, 1, _DIM_HEAD)
    bv2 = bv_c.reshape(_HEADS, 1, _DIM_HEAD)

    ck, cv = pl.pallas_call(
        _comp_kernel,
        grid=(_HEADS,),
        in_specs=[
            pl.BlockSpec((1, _NC, _CBLOCK, _DIM_HEAD), lambda h: (h, 0, 0, 0)),
            pl.BlockSpec((1, _NC, _CBLOCK, _DIM_HEAD), lambda h: (h, 0, 0, 0)),
            pl.BlockSpec((1, _CBLOCK, _DIM_HEAD), lambda h: (h, 0, 0)),
            pl.BlockSpec((1, _CBLOCK, _DIM_HEAD), lambda h: (h, 0, 0)),
            pl.BlockSpec((1, _CBLOCK, _DIM_HEAD, _DIM_HEAD), lambda h: (h, 0, 0, 0)),
            pl.BlockSpec((1, _CBLOCK, _DIM_HEAD, _DIM_HEAD), lambda h: (h, 0, 0, 0)),
            pl.BlockSpec((1, 1, _DIM_HEAD), lambda h: (h, 0, 0)),
            pl.BlockSpec((1, 1, _DIM_HEAD), lambda h: (h, 0, 0)),
        ],
        out_specs=[
            pl.BlockSpec((1, _NC, _DIM_HEAD), lambda h: (h, 0, 0)),
            pl.BlockSpec((1, _NC, _DIM_HEAD), lambda h: (h, 0, 0)),
        ],
        out_shape=[
            jax.ShapeDtypeStruct((_HEADS, _NC, _DIM_HEAD), _f32),
            jax.ShapeDtypeStruct((_HEADS, _NC, _DIM_HEAD), _f32),
        ],
    )(k4, v4, k_pos, v_pos, wk_f, wv_f, bk2, bv2)

    pad_rows = _CKP - _NMEM - _NC
    ck_full = jnp.pad(jnp.concatenate([mem_kv[0], ck], axis=1),
                      ((0, 0), (0, pad_rows), (0, 0)))
    cv_full = jnp.pad(jnp.concatenate([mem_kv[1], cv], axis=1),
                      ((0, 0), (0, pad_rows), (0, 0)))

    attn = pl.pallas_call(
        _attn_kernel,
        grid=(_HEADS, _NQ),
        in_specs=[
            pl.BlockSpec((1, _QCHUNK, _DIM_HEAD), lambda h, i: (h, i, 0)),
            pl.BlockSpec((1, _SEQ, _DIM_HEAD), lambda h, i: (h, 0, 0)),
            pl.BlockSpec((1, _SEQ, _DIM_HEAD), lambda h, i: (h, 0, 0)),
            pl.BlockSpec((1, _CKP, _DIM_HEAD), lambda h, i: (h, 0, 0)),
            pl.BlockSpec((1, _CKP, _DIM_HEAD), lambda h, i: (h, 0, 0)),
            pl.BlockSpec((_QCHUNK, _GPAD), lambda h, i: (i, 0)),
        ],
        out_specs=pl.BlockSpec((1, _QCHUNK, _DIM_HEAD), lambda h, i: (h, i, 0)),
        out_shape=jax.ShapeDtypeStruct((_HEADS, _SEQ, _DIM_HEAD), _f32),
    )(q, k, v, ck_full, cv_full, comb)

    merged = attn.transpose(1, 0, 2).reshape(_SEQ, _DIM_INNER)

    out = pl.pallas_call(
        _proj_kernel,
        grid=(_NQ,),
        in_specs=[
            pl.BlockSpec((_QCHUNK, _DIM_INNER), lambda i: (i, 0)),
            pl.BlockSpec((_DIM_INNER, _DIM), lambda i: (0, 0)),
        ],
        out_specs=pl.BlockSpec((_QCHUNK, _DIM), lambda i: (i, 0)),
        out_shape=jax.ShapeDtypeStruct((_SEQ, _DIM), _f32),
    )(merged, W_out.T)

    return out[None]


# speedup vs baseline: 8.0360x; 2.5562x over previous
from functools import partial
from math import ceil

import jax
import jax.numpy as jnp
from jax.experimental import pallas as pl

_DIM = 768
_DIM_HEAD = 64
_HEADS = 12
_WINDOW = 32
_CBLOCK = 32
_SBLOCK = 32
_NSEL = 2
_NMEM = 4
_SEQ = 2048
_DIM_INNER = _DIM_HEAD * _HEADS

_QCHUNK = 256
_KCHUNK = 256
_NQ = _SEQ // _QCHUNK
_NC = _SEQ // _CBLOCK
_CKP = 128
_GPAD = 128

_MASKVAL = -jnp.finfo(jnp.float32).max
_NEGFILL = -3.0e38
_EPS = jnp.finfo(jnp.float32).eps
_SCALE = _DIM_HEAD ** -0.5

_f32 = jnp.float32


def _dot(a, b):
    return jnp.dot(a, b, preferred_element_type=_f32)


def _dot_nt(a, b):
    return jax.lax.dot_general(a, b, (((1,), (1,)), ((), ())),
                               preferred_element_type=_f32)


def _pre_kernel(inp_ref, nw_ref, wqkv_ref, wcomb_ref, bcomb_ref,
                qkv_ref, comb_ref):
    x = inp_ref[...]
    xn = x * jax.lax.rsqrt(jnp.mean(x * x, axis=-1, keepdims=True) + _EPS)
    xn = xn * nw_ref[...]
    qkv_ref[...] = _dot(xn, wqkv_ref[...])
    comb_ref[...] = jax.nn.sigmoid(_dot(xn, wcomb_ref[...]) + bcomb_ref[...])


def _comp_kernel(k_ref, v_ref, kpos_ref, vpos_ref, wk_ref, wv_ref,
                 bk_ref, bv_ref, ck_ref, cv_ref, vmean_ref):
    acc_k = jnp.zeros((_NC, _DIM_HEAD), _f32)
    acc_v = jnp.zeros((_NC, _DIM_HEAD), _f32)
    acc_m = jnp.zeros((_NC, _DIM_HEAD), _f32)
    for t in range(_CBLOCK):
        raw_v = v_ref[0, :, t, :]
        kk_t = k_ref[0, :, t, :] + kpos_ref[0, t, :][None]
        vv_t = raw_v + vpos_ref[0, t, :][None]
        acc_k = acc_k + _dot(kk_t, wk_ref[0, t])
        acc_v = acc_v + _dot(vv_t, wv_ref[0, t])
        acc_m = acc_m + raw_v
    ck_ref[0] = acc_k + bk_ref[0]
    cv_ref[0] = acc_v + bv_ref[0]
    vmean_ref[0] = acc_m * (1.0 / _CBLOCK)


def _attn_kernel(q_ref, kp_ref, kc_ref, vp_ref, vc_ref, k0_ref, v0_ref,
                 ck_ref, cv_ref, vm_ref, comb_ref, out_ref):
    h = pl.program_id(0)
    qi = pl.program_id(1)
    q = q_ref[0] * _SCALE
    rows = qi * _QCHUNK + jax.lax.broadcasted_iota(jnp.int32, (_QCHUNK, 1), 0)
    colsj = jax.lax.broadcasted_iota(jnp.int32, (1, _CKP), 1)

    csim = _dot_nt(q, ck_ref[0])
    ck_seq = jnp.where(colsj < _NMEM, -1, (colsj - _NMEM + 1) * _CBLOCK - 1)
    cmask = (ck_seq < rows) | (colsj >= _NMEM + _NC)
    csim = jnp.where(cmask, _MASKVAL, csim)
    cm = jnp.max(csim, axis=-1, keepdims=True)
    cp = jnp.exp(csim - cm)
    cl = jnp.sum(cp, axis=-1, keepdims=True)
    cmp_out = _dot(cp, cv_ref[0]) / cl

    imp = jnp.where((colsj >= _NMEM) & (colsj < _NMEM + _NC), csim, -jnp.inf)
    big = jnp.int32(1 << 20)
    m1 = jnp.max(imp, axis=-1, keepdims=True)
    a1 = jnp.min(jnp.where(imp == m1, colsj, big), axis=-1, keepdims=True)
    imp2 = jnp.where(colsj == a1, -jnp.inf, imp)
    m2 = jnp.max(imp2, axis=-1, keepdims=True)
    a2 = jnp.min(jnp.where(imp2 == m2, colsj, big), axis=-1, keepdims=True)
    sel0 = a1 - _NMEM
    sel1 = a2 - _NMEM
    v1 = m2 > _MASKVAL

    wcols = jax.lax.broadcasted_iota(jnp.int32, (1, _NC), 1)
    oh = (jnp.where(wcols == sel0, 0.5, 0.0)
          + jnp.where(wcols == sel1, 0.5, 0.0))
    fine_u = _dot(oh, vm_ref[0])
    s0 = _dot_nt(q, k0_ref[0])
    p0 = jnp.exp(s0 - jnp.max(s0, axis=-1, keepdims=True))
    f0 = _dot(p0, v0_ref[0]) / jnp.sum(p0, axis=-1, keepdims=True)
    fine_out = jnp.where(v1, fine_u, f0)

    sprev = _dot_nt(q, kp_ref[0])
    scur = _dot_nt(q, kc_ref[0])
    sl = jnp.concatenate([sprev, scur], axis=1)
    lane = jax.lax.broadcasted_iota(jnp.int32, (1, 2 * _KCHUNK), 1)
    pstart = jnp.maximum(qi - 1, 0) * _KCHUNK
    colg = jnp.where(lane < _KCHUNK, pstart + lane,
                     qi * _KCHUNK + lane - _KCHUNK)
    lallow = (colg <= rows) & (rows - colg <= _WINDOW)
    lallow = lallow & ((lane >= _KCHUNK) | (qi > 0))
    sl = jnp.where(lallow, sl, _NEGFILL)
    pll = jnp.exp(sl - jnp.max(sl, axis=-1, keepdims=True))
    vcat = jnp.concatenate([vp_ref[0], vc_ref[0]], axis=0)
    local_out = _dot(pll, vcat) / jnp.sum(pll, axis=-1, keepdims=True)

    comb = comb_ref[...]
    z = jnp.zeros_like(comb)
    g0 = jnp.sum(jnp.where(colsj == 3 * h, comb, z), axis=-1, keepdims=True)
    g1 = jnp.sum(jnp.where(colsj == 3 * h + 1, comb, z), axis=-1, keepdims=True)
    g2 = jnp.sum(jnp.where(colsj == 3 * h + 2, comb, z), axis=-1, keepdims=True)
    out_ref[0] = g0 * cmp_out + g1 * fine_out + g2 * local_out


def _proj_kernel(x_ref, w_ref, o_ref):
    o_ref[...] = _dot(x_ref[...], w_ref[...])


def kernel(inp, norm_w, W_qkv, mem_kv, k_pos, v_pos, Wk_c, bk_c, Wv_c, bv_c,
           W_comb, b_comb, W_out):
    x = inp[0]

    wqkv_t = W_qkv.T
    wcomb_t = jnp.pad(W_comb.T, ((0, 0), (0, _GPAD - 3 * _HEADS)))
    bcomb = jnp.pad(b_comb, (0, _GPAD - 3 * _HEADS))[None]

    qkv, comb = pl.pallas_call(
        _pre_kernel,
        grid=(_NQ,),
        in_specs=[
            pl.BlockSpec((_QCHUNK, _DIM), lambda i: (i, 0)),
            pl.BlockSpec((1, _DIM), lambda i: (0, 0)),
            pl.BlockSpec((_DIM, 3 * _DIM_INNER), lambda i: (0, 0)),
            pl.BlockSpec((_DIM, _GPAD), lambda i: (0, 0)),
            pl.BlockSpec((1, _GPAD), lambda i: (0, 0)),
        ],
        out_specs=[
            pl.BlockSpec((_QCHUNK, 3 * _DIM_INNER), lambda i: (i, 0)),
            pl.BlockSpec((_QCHUNK, _GPAD), lambda i: (i, 0)),
        ],
        out_shape=[
            jax.ShapeDtypeStruct((_SEQ, 3 * _DIM_INNER), _f32),
            jax.ShapeDtypeStruct((_SEQ, _GPAD), _f32),
        ],
    )(x, norm_w[None], wqkv_t, wcomb_t, bcomb)

    q = qkv[:, :_DIM_INNER].reshape(_SEQ, _HEADS, _DIM_HEAD).transpose(1, 0, 2)
    k = qkv[:, _DIM_INNER:2 * _DIM_INNER].reshape(_SEQ, _HEADS, _DIM_HEAD).transpose(1, 0, 2)
    v = qkv[:, 2 * _DIM_INNER:].reshape(_SEQ, _HEADS, _DIM_HEAD).transpose(1, 0, 2)

    k4 = k.reshape(_HEADS, _NC, _CBLOCK, _DIM_HEAD)
    v4 = v.reshape(_HEADS, _NC, _CBLOCK, _DIM_HEAD)
    wk_f = Wk_c.reshape(_HEADS, _DIM_HEAD, _DIM_HEAD, _CBLOCK).transpose(0, 3, 2, 1)
    wv_f = Wv_c.reshape(_HEADS, _DIM_HEAD, _DIM_HEAD, _CBLOCK).transpose(0, 3, 2, 1)
    bk2 = bk_c.reshape(_HEADS, 1, _DIM_HEAD)
    bv2 = bv_c.reshape(_HEADS, 1, _DIM_HEAD)

    ck, cv, vmean = pl.pallas_call(
        _comp_kernel,
        grid=(_HEADS,),
        in_specs=[
            pl.BlockSpec((1, _NC, _CBLOCK, _DIM_HEAD), lambda h: (h, 0, 0, 0)),
            pl.BlockSpec((1, _NC, _CBLOCK, _DIM_HEAD), lambda h: (h, 0, 0, 0)),
            pl.BlockSpec((1, _CBLOCK, _DIM_HEAD), lambda h: (h, 0, 0)),
            pl.BlockSpec((1, _CBLOCK, _DIM_HEAD), lambda h: (h, 0, 0)),
            pl.BlockSpec((1, _CBLOCK, _DIM_HEAD, _DIM_HEAD), lambda h: (h, 0, 0, 0)),
            pl.BlockSpec((1, _CBLOCK, _DIM_HEAD, _DIM_HEAD), lambda h: (h, 0, 0, 0)),
            pl.BlockSpec((1, 1, _DIM_HEAD), lambda h: (h, 0, 0)),
            pl.BlockSpec((1, 1, _DIM_HEAD), lambda h: (h, 0, 0)),
        ],
        out_specs=[
            pl.BlockSpec((1, _NC, _DIM_HEAD), lambda h: (h, 0, 0)),
            pl.BlockSpec((1, _NC, _DIM_HEAD), lambda h: (h, 0, 0)),
            pl.BlockSpec((1, _NC, _DIM_HEAD), lambda h: (h, 0, 0)),
        ],
        out_shape=[
            jax.ShapeDtypeStruct((_HEADS, _NC, _DIM_HEAD), _f32),
            jax.ShapeDtypeStruct((_HEADS, _NC, _DIM_HEAD), _f32),
            jax.ShapeDtypeStruct((_HEADS, _NC, _DIM_HEAD), _f32),
        ],
    )(k4, v4, k_pos, v_pos, wk_f, wv_f, bk2, bv2)

    pad_rows = _CKP - _NMEM - _NC
    ck_full = jnp.pad(jnp.concatenate([mem_kv[0], ck], axis=1),
                      ((0, 0), (0, pad_rows), (0, 0)))
    cv_full = jnp.pad(jnp.concatenate([mem_kv[1], cv], axis=1),
                      ((0, 0), (0, pad_rows), (0, 0)))

    attn = pl.pallas_call(
        _attn_kernel,
        grid=(_HEADS, _NQ),
        in_specs=[
            pl.BlockSpec((1, _QCHUNK, _DIM_HEAD), lambda h, i: (h, i, 0)),
            pl.BlockSpec((1, _KCHUNK, _DIM_HEAD),
                         lambda h, i: (h, jnp.maximum(i - 1, 0), 0)),
            pl.BlockSpec((1, _KCHUNK, _DIM_HEAD), lambda h, i: (h, i, 0)),
            pl.BlockSpec((1, _KCHUNK, _DIM_HEAD),
                         lambda h, i: (h, jnp.maximum(i - 1, 0), 0)),
            pl.BlockSpec((1, _KCHUNK, _DIM_HEAD), lambda h, i: (h, i, 0)),
            pl.BlockSpec((1, _CBLOCK, _DIM_HEAD), lambda h, i: (h, 0, 0)),
            pl.BlockSpec((1, _CBLOCK, _DIM_HEAD), lambda h, i: (h, 0, 0)),
            pl.BlockSpec((1, _CKP, _DIM_HEAD), lambda h, i: (h, 0, 0)),
            pl.BlockSpec((1, _CKP, _DIM_HEAD), lambda h, i: (h, 0, 0)),
            pl.BlockSpec((1, _NC, _DIM_HEAD), lambda h, i: (h, 0, 0)),
            pl.BlockSpec((_QCHUNK, _GPAD), lambda h, i: (i, 0)),
        ],
        out_specs=pl.BlockSpec((1, _QCHUNK, _DIM_HEAD), lambda h, i: (h, i, 0)),
        out_shape=jax.ShapeDtypeStruct((_HEADS, _SEQ, _DIM_HEAD), _f32),
    )(q, k, k, v, v, k, v, ck_full, cv_full, vmean, comb)

    merged = attn.transpose(1, 0, 2).reshape(_SEQ, _DIM_INNER)

    out = pl.pallas_call(
        _proj_kernel,
        grid=(_NQ,),
        in_specs=[
            pl.BlockSpec((_QCHUNK, _DIM_INNER), lambda i: (i, 0)),
            pl.BlockSpec((_DIM_INNER, _DIM), lambda i: (0, 0)),
        ],
        out_specs=pl.BlockSpec((_QCHUNK, _DIM), lambda i: (i, 0)),
        out_shape=jax.ShapeDtypeStruct((_SEQ, _DIM), _f32),
    )(merged, W_out.T)

    return out[None]
